# baseline (speedup 1.0000x reference)
import functools

import jax
import jax.numpy as jnp
from jax import lax
from jax.experimental import pallas as pl
from jax.experimental.pallas import tpu as pltpu
from jax.experimental.pallas import tpu_sc as plsc

B, C, H, W = 8, 384, 14, 14
T = B * H * W
TQ = 2048
E, K, R = 8, 2, 4
DH = R * C
CAP = int(1.25 * T * K / E)
NCAP = 512
NQ = E * NCAP

_info = plsc.get_sparse_core_info()
NC, NS, L = _info.num_cores, _info.num_subcores, _info.num_lanes
NW = NC * NS
CHUNK = TQ // NW


def _f_body(x_ref, w_ref, dwb_ref, lnw_ref, lnb_ref, rw_ref,
            xpad_ref, res_ref, p1_ref, p2_ref, g1b_ref, g2b_ref, pad_ref):
    pad_ref[...] = jnp.zeros((H + 6, W + 6, B, C), jnp.float32)
    pad_ref[3:3 + H, 3:3 + W, :, :] = x_ref[...]
    acc = jnp.zeros((H, W, B, C), jnp.float32)
    for dh in range(7):
        for dw in range(7):
            wv = w_ref[dh * 7 + dw, :]
            acc = acc + pad_ref[dh:dh + H, dw:dw + W, :, :] * wv
    acc = acc + dwb_ref[...]
    mu = jnp.mean(acc, axis=-1, keepdims=True)
    xc = acc - mu
    var = jnp.mean(xc * xc, axis=-1, keepdims=True)
    xln = xc / jnp.sqrt(var + 1e-6) * lnw_ref[...] + lnb_ref[...]
    xs = xln.reshape(T, C)
    zpad = jnp.zeros((TQ - T, C), jnp.float32)
    xpad_ref[...] = jnp.concatenate([xs, zpad], axis=0)
    res_ref[...] = jnp.concatenate(
        [xs + x_ref[...].reshape(T, C), zpad], axis=0)
    logits = jnp.dot(xs, rw_ref[...], preferred_element_type=jnp.float32)
    lane = jax.lax.broadcasted_iota(jnp.int32, (T, E), 1)
    m = jnp.max(logits, axis=-1, keepdims=True)
    p = jnp.exp(logits - m)
    probs = p / jnp.sum(p, axis=-1, keepdims=True)
    m1 = jnp.max(probs, axis=-1, keepdims=True)
    i1t = jnp.min(jnp.where(probs == m1, lane, E), axis=-1, keepdims=True)
    pm = jnp.where(lane == i1t, -1.0, probs)
    m2 = jnp.max(pm, axis=-1, keepdims=True)
    i2t = jnp.min(jnp.where(pm == m2, lane, E), axis=-1, keepdims=True)
    ssum = m1 + m2
    colpad = jnp.zeros((TQ - T, 1), jnp.float32)
    pri = jnp.concatenate([m1, colpad - 1.0], axis=0)
    ipad = jnp.zeros((TQ - T, 1), jnp.int32)
    i1 = jnp.concatenate([i1t, ipad], axis=0)
    i2 = jnp.concatenate([i2t, ipad], axis=0)
    w1n = jnp.concatenate([m1 / ssum, colpad], axis=0)
    w2n = jnp.concatenate([m2 / ssum, colpad], axis=0)
    amat = jnp.concatenate(
        [((lane == i1t) | (lane == i2t)).astype(jnp.float32),
         jnp.zeros((TQ - T, E), jnp.float32)], axis=0)
    prow = jnp.transpose(pri, (1, 0))
    cparts = []
    for rb in range(TQ // 128):
        r0 = rb * 128
        pcol = pri[r0:r0 + 128, :]
        tcol = jax.lax.broadcasted_iota(jnp.int32, (128, TQ), 1)
        trow = jax.lax.broadcasted_iota(jnp.int32, (128, TQ), 0) + r0
        gt = prow > pcol
        eq = (prow == pcol) & (tcol < trow)
        mblk = jnp.where(gt | eq, 1.0, 0.0)
        cparts.append(jnp.dot(mblk, amat,
                              preferred_element_type=jnp.float32))
    cnt = jnp.concatenate(cparts, axis=0)
    lan2 = jax.lax.broadcasted_iota(jnp.int32, (TQ, E), 1)
    r1 = jnp.sum(jnp.where(lan2 == i1, cnt, 0.0), axis=-1, keepdims=True)
    r2 = jnp.sum(jnp.where(lan2 == i2, cnt, 0.0), axis=-1, keepdims=True)
    vrow = jax.lax.broadcasted_iota(jnp.int32, (TQ, 1), 0) < T
    k1 = (r1 < CAP) & vrow
    k2 = (r2 < CAP) & vrow
    ones16 = jnp.ones((1, 16), jnp.float32)
    g1b_ref[...] = (w1n * k1.astype(jnp.float32)) * ones16
    g2b_ref[...] = (w2n * k2.astype(jnp.float32)) * ones16
    p1_ref[...] = jnp.where(k1, i1 * NCAP + r1.astype(jnp.int32), NQ - 1)
    p2_ref[...] = jnp.where(k2, i2 * NCAP + r2.astype(jnp.int32), NQ - 1)


_mesh = plsc.VectorSubcoreMesh(core_axis_name="c", subcore_axis_name="s")


@functools.partial(
    pl.kernel, mesh=_mesh,
    out_type=jax.ShapeDtypeStruct((NQ, C), jnp.float32),
    scratch_types=[
        pltpu.VMEM((CHUNK, C), jnp.float32),
        pltpu.VMEM((CHUNK,), jnp.int32),
        pltpu.VMEM((CHUNK,), jnp.int32),
        pltpu.SemaphoreType.DMA,
    ],
)
def _sc_dispatch(x_hbm, p1_hbm, p2_hbm, d_hbm, xbuf, i1b, i2b, sem):
    wid = lax.axis_index("s") * NC + lax.axis_index("c")
    t0 = wid * CHUNK
    pltpu.sync_copy(p1_hbm.at[pl.ds(t0, CHUNK)], i1b)
    pltpu.sync_copy(p2_hbm.at[pl.ds(t0, CHUNK)], i2b)
    pltpu.sync_copy(x_hbm.at[pl.ds(t0, CHUNK)], xbuf)
    pltpu.async_copy(xbuf, d_hbm.at[i1b], sem).wait()
    pltpu.async_copy(xbuf, d_hbm.at[i2b], sem).wait()


@functools.partial(
    pl.kernel, mesh=_mesh,
    out_type=jax.ShapeDtypeStruct((TQ, C), jnp.float32),
    scratch_types=[
        pltpu.VMEM((CHUNK, C), jnp.float32),
        pltpu.VMEM((CHUNK, C), jnp.float32),
        pltpu.VMEM((CHUNK, C), jnp.float32),
        pltpu.VMEM((CHUNK,), jnp.int32),
        pltpu.VMEM((CHUNK,), jnp.int32),
        pltpu.VMEM((CHUNK, 16), jnp.float32),
        pltpu.VMEM((CHUNK, 16), jnp.float32),
        pltpu.SemaphoreType.DMA,
    ],
)
def _sc_combine(y_hbm, p1_hbm, p2_hbm, g1_hbm, g2_hbm, r_hbm, o_hbm,
                b1v, b2v, rbuf, i1b, i2b, g1buf, g2buf, sem):
    wid = lax.axis_index("s") * NC + lax.axis_index("c")
    t0 = wid * CHUNK
    pltpu.sync_copy(p1_hbm.at[pl.ds(t0, CHUNK)], i1b)
    pltpu.sync_copy(p2_hbm.at[pl.ds(t0, CHUNK)], i2b)
    pltpu.async_copy(y_hbm.at[i1b], b1v, sem).wait()
    pltpu.async_copy(y_hbm.at[i2b], b2v, sem).wait()
    pltpu.sync_copy(r_hbm.at[pl.ds(t0, CHUNK)], rbuf)
    pltpu.sync_copy(g1_hbm.at[pl.ds(t0, CHUNK)], g1buf)
    pltpu.sync_copy(g2_hbm.at[pl.ds(t0, CHUNK)], g2buf)

    def body(t, carry):
        gv1 = g1buf[t, :]
        gv2 = g2buf[t, :]
        for j in range(C // 16):
            sl = pl.ds(j * 16, 16)
            rbuf[t, sl] = rbuf[t, sl] + b1v[t, sl] * gv1 + b2v[t, sl] * gv2
        return carry

    lax.fori_loop(0, CHUNK, body, 0)
    pltpu.sync_copy(rbuf, o_hbm.at[pl.ds(t0, CHUNK)])


def _cd_body(d_ref, w1_ref, b1_ref, w2_ref, b2_ref, ls_ref, y_ref):
    d = d_ref[...].astype(jnp.bfloat16)
    h = jnp.dot(d, w1_ref[0].astype(jnp.bfloat16),
                preferred_element_type=jnp.float32) + b1_ref[0]
    g = 0.5 * h * (1.0 + jax.lax.erf(h * 0.7071067811865476))
    y = jnp.dot(g.astype(jnp.bfloat16), w2_ref[0].astype(jnp.bfloat16),
                preferred_element_type=jnp.float32) + b2_ref[0]
    y_ref[...] = y * ls_ref[...]


def kernel(input, dw_w, dw_b, ln_w, ln_b, router_w, w1, b1, w2, b2,
           layer_scale):
    f32 = jnp.float32
    x_t = jnp.transpose(input, (2, 3, 0, 1))
    wconv = jnp.transpose(dw_w[:, 0], (1, 2, 0)).reshape(49, C)

    col_i = jax.ShapeDtypeStruct((TQ, 1), jnp.int32)
    lanes = jax.ShapeDtypeStruct((TQ, 16), f32)
    xpad, resid, pos1, pos2, g1b, g2b = pl.pallas_call(
        _f_body,
        out_shape=[jax.ShapeDtypeStruct((TQ, C), f32),
                   jax.ShapeDtypeStruct((TQ, C), f32),
                   col_i, col_i, lanes, lanes],
        scratch_shapes=[pltpu.VMEM((H + 6, W + 6, B, C), f32)],
    )(x_t, wconv, dw_b, ln_w, ln_b, router_w)

    p1f = pos1.reshape(TQ)
    p2f = pos2.reshape(TQ)
    disp = _sc_dispatch(xpad, p1f, p2f)

    yout = pl.pallas_call(
        _cd_body,
        grid=(E,),
        in_specs=[
            pl.BlockSpec((NCAP, C), lambda e: (e, 0)),
            pl.BlockSpec((1, C, DH), lambda e: (e, 0, 0)),
            pl.BlockSpec((1, 1, DH), lambda e: (e, 0, 0)),
            pl.BlockSpec((1, DH, C), lambda e: (e, 0, 0)),
            pl.BlockSpec((1, 1, C), lambda e: (e, 0, 0)),
            pl.BlockSpec((1, C), lambda e: (0, 0)),
        ],
        out_specs=pl.BlockSpec((NCAP, C), lambda e: (e, 0)),
        out_shape=jax.ShapeDtypeStruct((NQ, C), f32),
    )(disp, w1, b1.reshape(E, 1, DH), w2, b2.reshape(E, 1, C),
      layer_scale.reshape(1, C))

    out_s = _sc_combine(yout, p1f, p2f, g1b, g2b, resid)

    out = out_s[:T].reshape(H, W, B, C)
    return jnp.transpose(out, (2, 3, 0, 1))


# speedup vs baseline: 2.4146x; 2.4146x over previous
import jax
import jax.numpy as jnp
from jax.experimental import pallas as pl
from jax.experimental.pallas import tpu as pltpu

B, C, H, W = 8, 384, 14, 14
T = B * H * W
TQ = 2048
TN = 1664
E, K, R = 8, 2, 4
DH = R * C
CAP = int(1.25 * T * K / E)
NCAP = 512
NQ = E * NCAP


def _mega_body(x_ref, w_ref, dwb_ref, lnw_ref, lnb_ref, rw_ref,
               w1_ref, b1_ref, w2_ref, b2_ref, ls_ref,
               o_ref,
               pad_ref, xbf_ref, p1_ref, p2_ref, g1_ref, g2_ref):
    s = pl.program_id(0)

    @pl.when(s == 0)
    def _front():
        pad_ref[...] = jnp.zeros((H + 6, W + 6, B, C), jnp.float32)
        pad_ref[3:3 + H, 3:3 + W, :, :] = x_ref[...]
        acc = jnp.zeros((H, W, B, C), jnp.float32)
        for dh in range(7):
            for dw in range(7):
                wv = w_ref[dh * 7 + dw, :]
                acc = acc + pad_ref[dh:dh + H, dw:dw + W, :, :] * wv
        acc = acc + dwb_ref[...]
        mu = jnp.mean(acc, axis=-1, keepdims=True)
        xc = acc - mu
        var = jnp.mean(xc * xc, axis=-1, keepdims=True)
        xln = xc / jnp.sqrt(var + 1e-6) * lnw_ref[...] + lnb_ref[...]
        xs = xln.reshape(T, C)
        zpad = jnp.zeros((TQ - T, C), jnp.float32)
        xp = jnp.concatenate([xs, zpad], axis=0)
        xbf_ref[...] = xp.astype(jnp.bfloat16)
        o_ref[...] = jnp.concatenate(
            [xs + x_ref[...].reshape(T, C), zpad], axis=0)
        logits = jnp.dot(xs, rw_ref[...],
                         preferred_element_type=jnp.float32)
        lane = jax.lax.broadcasted_iota(jnp.int32, (T, E), 1)
        m = jnp.max(logits, axis=-1, keepdims=True)
        p = jnp.exp(logits - m)
        probs = p / jnp.sum(p, axis=-1, keepdims=True)
        m1 = jnp.max(probs, axis=-1, keepdims=True)
        i1t = jnp.min(jnp.where(probs == m1, lane, E), axis=-1, keepdims=True)
        pm = jnp.where(lane == i1t, -1.0, probs)
        m2 = jnp.max(pm, axis=-1, keepdims=True)
        i2t = jnp.min(jnp.where(pm == m2, lane, E), axis=-1, keepdims=True)
        ssum = m1 + m2
        colpad = jnp.zeros((TQ - T, 1), jnp.float32)
        pri = jnp.concatenate([m1, colpad - 1.0], axis=0)
        ipad = jnp.zeros((TQ - T, 1), jnp.int32)
        i1 = jnp.concatenate([i1t, ipad], axis=0)
        i2 = jnp.concatenate([i2t, ipad], axis=0)
        w1n = jnp.concatenate([m1 / ssum, colpad], axis=0)
        w2n = jnp.concatenate([m2 / ssum, colpad], axis=0)
        amat = jnp.concatenate(
            [((lane == i1t) | (lane == i2t)).astype(jnp.float32),
             jnp.zeros((TQ - T, E), jnp.float32)], axis=0)
        prow = jnp.transpose(pri[0:TN, :], (1, 0))
        amn = amat[0:TN, :]
        cparts = []
        for rb in range(TN // 128):
            r0 = rb * 128
            pcol = pri[r0:r0 + 128, :]
            tcol = jax.lax.broadcasted_iota(jnp.int32, (128, TN), 1)
            trow = jax.lax.broadcasted_iota(jnp.int32, (128, TN), 0) + r0
            gt = prow > pcol
            eq = (prow == pcol) & (tcol < trow)
            mblk = jnp.where(gt | eq, 1.0, 0.0)
            cparts.append(jnp.dot(mblk, amn,
                                  preferred_element_type=jnp.float32))
        cparts.append(jnp.zeros((TQ - TN, E), jnp.float32))
        cnt = jnp.concatenate(cparts, axis=0)
        lan2 = jax.lax.broadcasted_iota(jnp.int32, (TQ, E), 1)
        r1 = jnp.sum(jnp.where(lan2 == i1, cnt, 0.0), axis=-1, keepdims=True)
        r2 = jnp.sum(jnp.where(lan2 == i2, cnt, 0.0), axis=-1, keepdims=True)
        vrow = jax.lax.broadcasted_iota(jnp.int32, (TQ, 1), 0) < T
        k1 = (r1 < CAP) & vrow
        k2 = (r2 < CAP) & vrow
        g1_ref[...] = w1n * k1.astype(jnp.float32)
        g2_ref[...] = w2n * k2.astype(jnp.float32)
        p1_ref[...] = jnp.where(k1, i1 * NCAP + r1.astype(jnp.int32), NQ - 1)
        p2_ref[...] = jnp.where(k2, i2 * NCAP + r2.astype(jnp.int32), NQ - 1)

    @pl.when(s > 0)
    def _expert():
        e = s - 1
        q = jax.lax.broadcasted_iota(jnp.int32, (TN, NCAP), 1) + e * NCAP
        mq1 = p1_ref[0:TN, :] == q
        mq2 = p2_ref[0:TN, :] == q
        qt = (mq1 | mq2).astype(jnp.bfloat16)
        d = jax.lax.dot_general(qt, xbf_ref[0:TN, :],
                                (((0,), (0,)), ((), ())),
                                preferred_element_type=jnp.float32)
        h = jnp.dot(d.astype(jnp.bfloat16), w1_ref[0].astype(jnp.bfloat16),
                    preferred_element_type=jnp.float32) + b1_ref[0]
        g = 0.5 * h * (1.0 + jax.lax.erf(h * 0.7071067811865476))
        y = jnp.dot(g.astype(jnp.bfloat16), w2_ref[0].astype(jnp.bfloat16),
                    preferred_element_type=jnp.float32) + b2_ref[0]
        y = (y * ls_ref[...]).astype(jnp.bfloat16)
        wc = (jnp.where(mq1, g1_ref[0:TN, :], 0.0)
              + jnp.where(mq2, g2_ref[0:TN, :], 0.0)).astype(jnp.bfloat16)
        o_ref[0:TN, :] += jnp.dot(wc, y, preferred_element_type=jnp.float32)


def kernel(input, dw_w, dw_b, ln_w, ln_b, router_w, w1, b1, w2, b2,
           layer_scale):
    f32 = jnp.float32
    x_t = jnp.transpose(input, (2, 3, 0, 1))
    wconv = jnp.transpose(dw_w[:, 0], (1, 2, 0)).reshape(49, C)

    ew = lambda s: pl.BlockSpec(s, lambda i: (jnp.maximum(i - 1, 0),)
                                + (0,) * (len(s) - 1))
    full = lambda s: pl.BlockSpec(s, lambda i: (0,) * len(s))
    out_s = pl.pallas_call(
        _mega_body,
        grid=(1 + E,),
        in_specs=[
            full((H, W, B, C)),
            full((49, C)),
            full((C,)),
            full((C,)),
            full((C,)),
            full((C, E)),
            ew((1, C, DH)),
            ew((1, 1, DH)),
            ew((1, DH, C)),
            ew((1, 1, C)),
            full((1, C)),
        ],
        out_specs=full((TQ, C)),
        out_shape=jax.ShapeDtypeStruct((TQ, C), f32),
        scratch_shapes=[
            pltpu.VMEM((H + 6, W + 6, B, C), f32),
            pltpu.VMEM((TQ, C), jnp.bfloat16),
            pltpu.VMEM((TQ, 1), jnp.int32),
            pltpu.VMEM((TQ, 1), jnp.int32),
            pltpu.VMEM((TQ, 1), f32),
            pltpu.VMEM((TQ, 1), f32),
        ],
    )(x_t, wconv, dw_b, ln_w, ln_b, router_w, w1, b1.reshape(E, 1, DH),
      w2, b2.reshape(E, 1, C), layer_scale.reshape(1, C))

    out = out_s[:T].reshape(H, W, B, C)
    return jnp.transpose(out, (2, 3, 0, 1))


# speedup vs baseline: 2.4191x; 1.0019x over previous
import jax
import jax.numpy as jnp
from jax.experimental import pallas as pl
from jax.experimental.pallas import tpu as pltpu

B, C, H, W = 8, 384, 14, 14
T = B * H * W
TQ = 1664
TN = 1664
E, K, R = 8, 2, 4
DH = R * C
CAP = int(1.25 * T * K / E)
NCAP = 512
NQ = E * NCAP


def _mega_body(x_ref, w_ref, dwb_ref, lnw_ref, lnb_ref, rw_ref,
               w1_ref, b1_ref, w2_ref, b2_ref, ls_ref,
               o_ref,
               pad_ref, xbf_ref, p1_ref, p2_ref, g1_ref, g2_ref):
    s = pl.program_id(0)

    @pl.when(s == 0)
    def _front():
        pad_ref[...] = jnp.zeros((H + 6, W + 6, B, C), jnp.float32)
        pad_ref[3:3 + H, 3:3 + W, :, :] = x_ref[...]
        acc = jnp.zeros((H, W, B, C), jnp.float32)
        for dh in range(7):
            for dw in range(7):
                wv = w_ref[dh * 7 + dw, :]
                acc = acc + pad_ref[dh:dh + H, dw:dw + W, :, :] * wv
        acc = acc + dwb_ref[...]
        mu = jnp.mean(acc, axis=-1, keepdims=True)
        xc = acc - mu
        var = jnp.mean(xc * xc, axis=-1, keepdims=True)
        xln = xc / jnp.sqrt(var + 1e-6) * lnw_ref[...] + lnb_ref[...]
        xs = xln.reshape(T, C)
        zpad = jnp.zeros((TQ - T, C), jnp.float32)
        xp = jnp.concatenate([xs, zpad], axis=0)
        xbf_ref[...] = xp.astype(jnp.bfloat16)
        o_ref[...] = jnp.concatenate(
            [xs + x_ref[...].reshape(T, C), zpad], axis=0)
        logits = jnp.dot(xs, rw_ref[...],
                         preferred_element_type=jnp.float32)
        lane = jax.lax.broadcasted_iota(jnp.int32, (T, E), 1)
        m = jnp.max(logits, axis=-1, keepdims=True)
        p = jnp.exp(logits - m)
        probs = p / jnp.sum(p, axis=-1, keepdims=True)
        m1 = jnp.max(probs, axis=-1, keepdims=True)
        i1t = jnp.min(jnp.where(probs == m1, lane, E), axis=-1, keepdims=True)
        pm = jnp.where(lane == i1t, -1.0, probs)
        m2 = jnp.max(pm, axis=-1, keepdims=True)
        i2t = jnp.min(jnp.where(pm == m2, lane, E), axis=-1, keepdims=True)
        ssum = m1 + m2
        colpad = jnp.zeros((TQ - T, 1), jnp.float32)
        pri = jnp.concatenate([m1, colpad - 1.0], axis=0)
        ipad = jnp.zeros((TQ - T, 1), jnp.int32)
        i1 = jnp.concatenate([i1t, ipad], axis=0)
        i2 = jnp.concatenate([i2t, ipad], axis=0)
        w1n = jnp.concatenate([m1 / ssum, colpad], axis=0)
        w2n = jnp.concatenate([m2 / ssum, colpad], axis=0)
        amat = jnp.concatenate(
            [((lane == i1t) | (lane == i2t)).astype(jnp.float32),
             jnp.zeros((TQ - T, E), jnp.float32)], axis=0)
        prow = jnp.transpose(pri[0:TN, :], (1, 0))
        amn = amat[0:TN, :]
        cparts = []
        for rb in range(TN // 128):
            r0 = rb * 128
            pcol = pri[r0:r0 + 128, :]
            tcol = jax.lax.broadcasted_iota(jnp.int32, (128, TN), 1)
            trow = jax.lax.broadcasted_iota(jnp.int32, (128, TN), 0) + r0
            gt = prow > pcol
            eq = (prow == pcol) & (tcol < trow)
            mblk = jnp.where(gt | eq, 1.0, 0.0)
            cparts.append(jnp.dot(mblk, amn,
                                  preferred_element_type=jnp.float32))
        cnt = jnp.concatenate(cparts, axis=0)
        lan2 = jax.lax.broadcasted_iota(jnp.int32, (TQ, E), 1)
        r1 = jnp.sum(jnp.where(lan2 == i1, cnt, 0.0), axis=-1, keepdims=True)
        r2 = jnp.sum(jnp.where(lan2 == i2, cnt, 0.0), axis=-1, keepdims=True)
        vrow = jax.lax.broadcasted_iota(jnp.int32, (TQ, 1), 0) < T
        k1 = (r1 < CAP) & vrow
        k2 = (r2 < CAP) & vrow
        g1_ref[...] = w1n * k1.astype(jnp.float32)
        g2_ref[...] = w2n * k2.astype(jnp.float32)
        p1_ref[...] = jnp.where(k1, i1 * NCAP + r1.astype(jnp.int32), NQ - 1)
        p2_ref[...] = jnp.where(k2, i2 * NCAP + r2.astype(jnp.int32), NQ - 1)

    @pl.when(s > 0)
    def _expert():
        e = s - 1
        q = jax.lax.broadcasted_iota(jnp.int32, (TN, NCAP), 1) + e * NCAP
        mq1 = p1_ref[0:TN, :] == q
        mq2 = p2_ref[0:TN, :] == q
        qt = (mq1 | mq2).astype(jnp.bfloat16)
        d = jax.lax.dot_general(qt, xbf_ref[0:TN, :],
                                (((0,), (0,)), ((), ())),
                                preferred_element_type=jnp.float32)
        h = jnp.dot(d.astype(jnp.bfloat16), w1_ref[0].astype(jnp.bfloat16),
                    preferred_element_type=jnp.float32) + b1_ref[0]
        g = 0.5 * h * (1.0 + jax.lax.erf(h * 0.7071067811865476))
        y = jnp.dot(g.astype(jnp.bfloat16), w2_ref[0].astype(jnp.bfloat16),
                    preferred_element_type=jnp.float32) + b2_ref[0]
        y = (y * ls_ref[...]).astype(jnp.bfloat16)
        wc = (jnp.where(mq1, g1_ref[0:TN, :], 0.0)
              + jnp.where(mq2, g2_ref[0:TN, :], 0.0)).astype(jnp.bfloat16)
        o_ref[0:TN, :] += jnp.dot(wc, y, preferred_element_type=jnp.float32)


def kernel(input, dw_w, dw_b, ln_w, ln_b, router_w, w1, b1, w2, b2,
           layer_scale):
    f32 = jnp.float32
    x_t = jnp.transpose(input, (2, 3, 0, 1))
    wconv = jnp.transpose(dw_w[:, 0], (1, 2, 0)).reshape(49, C)

    ew = lambda s: pl.BlockSpec(s, lambda i: (jnp.maximum(i - 1, 0),)
                                + (0,) * (len(s) - 1))
    full = lambda s: pl.BlockSpec(s, lambda i: (0,) * len(s))
    out_s = pl.pallas_call(
        _mega_body,
        grid=(1 + E,),
        in_specs=[
            full((H, W, B, C)),
            full((49, C)),
            full((C,)),
            full((C,)),
            full((C,)),
            full((C, E)),
            ew((1, C, DH)),
            ew((1, 1, DH)),
            ew((1, DH, C)),
            ew((1, 1, C)),
            full((1, C)),
        ],
        out_specs=full((TQ, C)),
        out_shape=jax.ShapeDtypeStruct((TQ, C), f32),
        scratch_shapes=[
            pltpu.VMEM((H + 6, W + 6, B, C), f32),
            pltpu.VMEM((TQ, C), jnp.bfloat16),
            pltpu.VMEM((TQ, 1), jnp.int32),
            pltpu.VMEM((TQ, 1), jnp.int32),
            pltpu.VMEM((TQ, 1), f32),
            pltpu.VMEM((TQ, 1), f32),
        ],
    )(x_t, wconv, dw_b, ln_w, ln_b, router_w, w1, b1.reshape(E, 1, DH),
      w2, b2.reshape(E, 1, C), layer_scale.reshape(1, C))

    out = out_s[:T].reshape(H, W, B, C)
    return jnp.transpose(out, (2, 3, 0, 1))


# speedup vs baseline: 2.4439x; 1.0102x over previous
import jax
import jax.numpy as jnp
from jax.experimental import pallas as pl
from jax.experimental.pallas import tpu as pltpu

B, C, H, W = 8, 384, 14, 14
T = B * H * W
TQ = 1664
TN = 1664
E, K, R = 8, 2, 4
DH = R * C
CAP = int(1.25 * T * K / E)
NCAP = 512
NQ = E * NCAP


def _mega_body(x_ref, w_ref, dwb_ref, lnw_ref, lnb_ref, rw_ref,
               w1_ref, b1_ref, w2_ref, b2_ref, ls_ref,
               o_ref,
               pad_ref, xbf_ref, p1_ref, p2_ref, g1_ref, g2_ref):
    s = pl.program_id(0)

    @pl.when(s == 0)
    def _front():
        pad_ref[...] = jnp.zeros((H + 6, W + 6, B, C), jnp.float32)
        pad_ref[3:3 + H, 3:3 + W, :, :] = x_ref[...]
        accs = [jnp.zeros((H, W, B, C), jnp.float32) for _ in range(4)]
        for i in range(49):
            dh, dw = i // 7, i % 7
            wv = w_ref[i, :]
            accs[i % 4] = accs[i % 4] + pad_ref[dh:dh + H, dw:dw + W, :, :] * wv
        acc = (accs[0] + accs[1]) + (accs[2] + accs[3]) + dwb_ref[...]
        mu = jnp.mean(acc, axis=-1, keepdims=True)
        xc = acc - mu
        var = jnp.mean(xc * xc, axis=-1, keepdims=True)
        xln = xc / jnp.sqrt(var + 1e-6) * lnw_ref[...] + lnb_ref[...]
        xs = xln.reshape(T, C)
        zpad = jnp.zeros((TQ - T, C), jnp.float32)
        xp = jnp.concatenate([xs, zpad], axis=0)
        xbf_ref[...] = xp.astype(jnp.bfloat16)
        o_ref[...] = jnp.concatenate(
            [xs + x_ref[...].reshape(T, C), zpad], axis=0)
        logits = jnp.dot(xs, rw_ref[...],
                         preferred_element_type=jnp.float32)
        lane = jax.lax.broadcasted_iota(jnp.int32, (T, E), 1)
        m = jnp.max(logits, axis=-1, keepdims=True)
        p = jnp.exp(logits - m)
        probs = p / jnp.sum(p, axis=-1, keepdims=True)
        m1 = jnp.max(probs, axis=-1, keepdims=True)
        i1t = jnp.min(jnp.where(probs == m1, lane, E), axis=-1, keepdims=True)
        pm = jnp.where(lane == i1t, -1.0, probs)
        m2 = jnp.max(pm, axis=-1, keepdims=True)
        i2t = jnp.min(jnp.where(pm == m2, lane, E), axis=-1, keepdims=True)
        ssum = m1 + m2
        colpad = jnp.zeros((TQ - T, 1), jnp.float32)
        pri = jnp.concatenate([m1, colpad - 1.0], axis=0)
        ipad = jnp.zeros((TQ - T, 1), jnp.int32)
        i1 = jnp.concatenate([i1t, ipad], axis=0)
        i2 = jnp.concatenate([i2t, ipad], axis=0)
        w1n = jnp.concatenate([m1 / ssum, colpad], axis=0)
        w2n = jnp.concatenate([m2 / ssum, colpad], axis=0)
        amat = jnp.concatenate(
            [((lane == i1t) | (lane == i2t)).astype(jnp.float32),
             jnp.zeros((TQ - T, E), jnp.float32)], axis=0)
        prow = jnp.transpose(pri[0:TN, :], (1, 0))
        amn = amat[0:TN, :]
        cparts = []
        for rb in range(TN // 128):
            r0 = rb * 128
            pcol = pri[r0:r0 + 128, :]
            tcol = jax.lax.broadcasted_iota(jnp.int32, (128, TN), 1)
            trow = jax.lax.broadcasted_iota(jnp.int32, (128, TN), 0) + r0
            gt = prow > pcol
            eq = (prow == pcol) & (tcol < trow)
            mblk = jnp.where(gt | eq, 1.0, 0.0)
            cparts.append(jnp.dot(mblk, amn,
                                  preferred_element_type=jnp.float32))
        cnt = jnp.concatenate(cparts, axis=0)
        lan2 = jax.lax.broadcasted_iota(jnp.int32, (TQ, E), 1)
        r1 = jnp.sum(jnp.where(lan2 == i1, cnt, 0.0), axis=-1, keepdims=True)
        r2 = jnp.sum(jnp.where(lan2 == i2, cnt, 0.0), axis=-1, keepdims=True)
        vrow = jax.lax.broadcasted_iota(jnp.int32, (TQ, 1), 0) < T
        k1 = (r1 < CAP) & vrow
        k2 = (r2 < CAP) & vrow
        g1_ref[...] = w1n * k1.astype(jnp.float32)
        g2_ref[...] = w2n * k2.astype(jnp.float32)
        p1_ref[...] = jnp.where(k1, i1 * NCAP + r1.astype(jnp.int32), NQ - 1)
        p2_ref[...] = jnp.where(k2, i2 * NCAP + r2.astype(jnp.int32), NQ - 1)

    @pl.when(s > 0)
    def _expert():
        e = s - 1
        q = jax.lax.broadcasted_iota(jnp.int32, (TN, NCAP), 1) + e * NCAP
        mq1 = p1_ref[0:TN, :] == q
        mq2 = p2_ref[0:TN, :] == q
        qt = (mq1 | mq2).astype(jnp.bfloat16)
        d = jax.lax.dot_general(qt, xbf_ref[0:TN, :],
                                (((0,), (0,)), ((), ())),
                                preferred_element_type=jnp.float32)
        h = jnp.dot(d.astype(jnp.bfloat16), w1_ref[0].astype(jnp.bfloat16),
                    preferred_element_type=jnp.float32) + b1_ref[0]
        g = 0.5 * h * (1.0 + jax.lax.erf(h * 0.7071067811865476))
        y = jnp.dot(g.astype(jnp.bfloat16), w2_ref[0].astype(jnp.bfloat16),
                    preferred_element_type=jnp.float32) + b2_ref[0]
        y = (y * ls_ref[...]).astype(jnp.bfloat16)
        wc = (jnp.where(mq1, g1_ref[0:TN, :], 0.0)
              + jnp.where(mq2, g2_ref[0:TN, :], 0.0)).astype(jnp.bfloat16)
        o_ref[0:TN, :] += jnp.dot(wc, y, preferred_element_type=jnp.float32)


def kernel(input, dw_w, dw_b, ln_w, ln_b, router_w, w1, b1, w2, b2,
           layer_scale):
    f32 = jnp.float32
    x_t = jnp.transpose(input, (2, 3, 0, 1))
    wconv = jnp.transpose(dw_w[:, 0], (1, 2, 0)).reshape(49, C)

    ew = lambda s: pl.BlockSpec(s, lambda i: (jnp.maximum(i - 1, 0),)
                                + (0,) * (len(s) - 1))
    full = lambda s: pl.BlockSpec(s, lambda i: (0,) * len(s))
    out_s = pl.pallas_call(
        _mega_body,
        grid=(1 + E,),
        in_specs=[
            full((H, W, B, C)),
            full((49, C)),
            full((C,)),
            full((C,)),
            full((C,)),
            full((C, E)),
            ew((1, C, DH)),
            ew((1, 1, DH)),
            ew((1, DH, C)),
            ew((1, 1, C)),
            full((1, C)),
        ],
        out_specs=full((TQ, C)),
        out_shape=jax.ShapeDtypeStruct((TQ, C), f32),
        scratch_shapes=[
            pltpu.VMEM((H + 6, W + 6, B, C), f32),
            pltpu.VMEM((TQ, C), jnp.bfloat16),
            pltpu.VMEM((TQ, 1), jnp.int32),
            pltpu.VMEM((TQ, 1), jnp.int32),
            pltpu.VMEM((TQ, 1), f32),
            pltpu.VMEM((TQ, 1), f32),
        ],
    )(x_t, wconv, dw_b, ln_w, ln_b, router_w, w1, b1.reshape(E, 1, DH),
      w2, b2.reshape(E, 1, C), layer_scale.reshape(1, C))

    out = out_s[:T].reshape(H, W, B, C)
    return jnp.transpose(out, (2, 3, 0, 1))


# speedup vs baseline: 2.4718x; 1.0114x over previous
import jax
import jax.numpy as jnp
from jax.experimental import pallas as pl
from jax.experimental.pallas import tpu as pltpu

B, C, H, W = 8, 384, 14, 14
T = B * H * W
TQ = 1664
TN = 1664
E, K, R = 8, 2, 4
DH = R * C
CAP = int(1.25 * T * K / E)
NCAP = 512
NQ = E * NCAP


def _mega_body(x_ref, w_ref, dwb_ref, lnw_ref, lnb_ref, rw_ref,
               w1_ref, b1_ref, w2_ref, b2_ref, ls_ref,
               o_ref,
               pad_ref, xbf_ref, p1_ref, p2_ref, g1_ref, g2_ref, ybuf_ref):
    s = pl.program_id(0)

    @pl.when(s == 0)
    def _front():
        pad_ref[...] = jnp.zeros((H + 6, W + 6, B, C), jnp.float32)
        pad_ref[3:3 + H, 3:3 + W, :, :] = x_ref[...]
        accs = [jnp.zeros((H, W, B, C), jnp.float32) for _ in range(4)]
        for i in range(49):
            dh, dw = i // 7, i % 7
            wv = w_ref[i, :]
            accs[i % 4] = accs[i % 4] + pad_ref[dh:dh + H, dw:dw + W, :, :] * wv
        acc = (accs[0] + accs[1]) + (accs[2] + accs[3]) + dwb_ref[...]
        mu = jnp.mean(acc, axis=-1, keepdims=True)
        xc = acc - mu
        var = jnp.mean(xc * xc, axis=-1, keepdims=True)
        xln = xc / jnp.sqrt(var + 1e-6) * lnw_ref[...] + lnb_ref[...]
        xs = xln.reshape(T, C)
        zpad = jnp.zeros((TQ - T, C), jnp.float32)
        xp = jnp.concatenate([xs, zpad], axis=0)
        xbf_ref[...] = xp.astype(jnp.bfloat16)
        o_ref[...] = jnp.concatenate(
            [xs + x_ref[...].reshape(T, C), zpad], axis=0)
        logits = jnp.dot(xs, rw_ref[...],
                         preferred_element_type=jnp.float32)
        lane = jax.lax.broadcasted_iota(jnp.int32, (T, E), 1)
        m = jnp.max(logits, axis=-1, keepdims=True)
        p = jnp.exp(logits - m)
        probs = p / jnp.sum(p, axis=-1, keepdims=True)
        m1 = jnp.max(probs, axis=-1, keepdims=True)
        i1t = jnp.min(jnp.where(probs == m1, lane, E), axis=-1, keepdims=True)
        pm = jnp.where(lane == i1t, -1.0, probs)
        m2 = jnp.max(pm, axis=-1, keepdims=True)
        i2t = jnp.min(jnp.where(pm == m2, lane, E), axis=-1, keepdims=True)
        ssum = m1 + m2
        colpad = jnp.zeros((TQ - T, 1), jnp.float32)
        pri = jnp.concatenate([m1, colpad - 1.0], axis=0)
        ipad = jnp.zeros((TQ - T, 1), jnp.int32)
        i1 = jnp.concatenate([i1t, ipad], axis=0)
        i2 = jnp.concatenate([i2t, ipad], axis=0)
        w1n = jnp.concatenate([m1 / ssum, colpad], axis=0)
        w2n = jnp.concatenate([m2 / ssum, colpad], axis=0)
        amat = jnp.concatenate(
            [((lane == i1t) | (lane == i2t)).astype(jnp.float32),
             jnp.zeros((TQ - T, E), jnp.float32)], axis=0)
        prow = jnp.transpose(pri[0:TN, :], (1, 0))
        amn = amat[0:TN, :]
        cparts = []
        for rb in range(TN // 128):
            r0 = rb * 128
            pcol = pri[r0:r0 + 128, :]
            tcol = jax.lax.broadcasted_iota(jnp.int32, (128, TN), 1)
            trow = jax.lax.broadcasted_iota(jnp.int32, (128, TN), 0) + r0
            gt = prow > pcol
            eq = (prow == pcol) & (tcol < trow)
            mblk = jnp.where(gt | eq, 1.0, 0.0)
            cparts.append(jnp.dot(mblk, amn,
                                  preferred_element_type=jnp.float32))
        cnt = jnp.concatenate(cparts, axis=0)
        lan2 = jax.lax.broadcasted_iota(jnp.int32, (TQ, E), 1)
        r1 = jnp.sum(jnp.where(lan2 == i1, cnt, 0.0), axis=-1, keepdims=True)
        r2 = jnp.sum(jnp.where(lan2 == i2, cnt, 0.0), axis=-1, keepdims=True)
        vrow = jax.lax.broadcasted_iota(jnp.int32, (TQ, 1), 0) < T
        k1 = (r1 < CAP) & vrow
        k2 = (r2 < CAP) & vrow
        g1_ref[...] = w1n * k1.astype(jnp.float32)
        g2_ref[...] = w2n * k2.astype(jnp.float32)
        p1_ref[...] = jnp.where(k1, i1 * NCAP + r1.astype(jnp.int32), NQ - 1)
        p2_ref[...] = jnp.where(k2, i2 * NCAP + r2.astype(jnp.int32), NQ - 1)

    @pl.when((s > 0) & (s <= E))
    def _expert():
        e = s - 1
        q = jax.lax.broadcasted_iota(jnp.int32, (TN, NCAP), 1) + e * NCAP
        qt = ((p1_ref[0:TN, :] == q)
              | (p2_ref[0:TN, :] == q)).astype(jnp.bfloat16)
        d = jax.lax.dot_general(qt, xbf_ref[0:TN, :],
                                (((0,), (0,)), ((), ())),
                                preferred_element_type=jnp.float32)
        h = jnp.dot(d.astype(jnp.bfloat16), w1_ref[0].astype(jnp.bfloat16),
                    preferred_element_type=jnp.float32) + b1_ref[0]
        g = 0.5 * h * (1.0 + jax.lax.erf(h * 0.7071067811865476))
        y = jnp.dot(g.astype(jnp.bfloat16), w2_ref[0].astype(jnp.bfloat16),
                    preferred_element_type=jnp.float32) + b2_ref[0]
        ybuf_ref[pl.ds(e * NCAP, NCAP), :] = (
            (y * ls_ref[...]).astype(jnp.bfloat16))

    @pl.when(s == E + 1)
    def _combine():
        yv = ybuf_ref[...]
        for rb in range(TN // 128):
            r0 = rb * 128
            q = jax.lax.broadcasted_iota(jnp.int32, (128, NQ), 1)
            p1 = p1_ref[r0:r0 + 128, :]
            p2 = p2_ref[r0:r0 + 128, :]
            wc = (jnp.where(q == p1, g1_ref[r0:r0 + 128, :], 0.0)
                  + jnp.where(q == p2, g2_ref[r0:r0 + 128, :], 0.0)
                  ).astype(jnp.bfloat16)
            o_ref[r0:r0 + 128, :] += jnp.dot(
                wc, yv, preferred_element_type=jnp.float32)


def kernel(input, dw_w, dw_b, ln_w, ln_b, router_w, w1, b1, w2, b2,
           layer_scale):
    f32 = jnp.float32
    x_t = jnp.transpose(input, (2, 3, 0, 1))
    wconv = jnp.transpose(dw_w[:, 0], (1, 2, 0)).reshape(49, C)

    ew = lambda s: pl.BlockSpec(
        s, lambda i: (jnp.minimum(jnp.maximum(i - 1, 0), E - 1),)
        + (0,) * (len(s) - 1))
    full = lambda s: pl.BlockSpec(s, lambda i: (0,) * len(s))
    out_s = pl.pallas_call(
        _mega_body,
        grid=(2 + E,),
        in_specs=[
            full((H, W, B, C)),
            full((49, C)),
            full((C,)),
            full((C,)),
            full((C,)),
            full((C, E)),
            ew((1, C, DH)),
            ew((1, 1, DH)),
            ew((1, DH, C)),
            ew((1, 1, C)),
            full((1, C)),
        ],
        out_specs=full((TQ, C)),
        out_shape=jax.ShapeDtypeStruct((TQ, C), f32),
        scratch_shapes=[
            pltpu.VMEM((H + 6, W + 6, B, C), f32),
            pltpu.VMEM((TQ, C), jnp.bfloat16),
            pltpu.VMEM((TQ, 1), jnp.int32),
            pltpu.VMEM((TQ, 1), jnp.int32),
            pltpu.VMEM((TQ, 1), f32),
            pltpu.VMEM((TQ, 1), f32),
            pltpu.VMEM((NQ, C), jnp.bfloat16),
        ],
    )(x_t, wconv, dw_b, ln_w, ln_b, router_w, w1, b1.reshape(E, 1, DH),
      w2, b2.reshape(E, 1, C), layer_scale.reshape(1, C))

    out = out_s[:T].reshape(H, W, B, C)
    return jnp.transpose(out, (2, 3, 0, 1))
